# baseline (speedup 1.0000x reference)
import functools

import jax
import jax.numpy as jnp
from jax import lax
from jax.experimental import pallas as pl
from jax.experimental.pallas import tpu as pltpu
from jax.experimental.pallas import tpu_sc as plsc

N_NODES = 10000
N_EDGES = 320000
D = 128
THRESH = 0.005

NC = 2
NS = 16
NW = NC * NS
CH = 128
NCHUNK = 80
E_PAD = NW * NCHUNK * CH
N_PAD = 10240

_mesh = plsc.VectorSubcoreMesh(core_axis_name="c", subcore_axis_name="s")

_ROWS_PER_TILE = N_PAD // NS


@functools.partial(
    pl.kernel,
    out_type=[
        jax.ShapeDtypeStruct((NC, N_PAD), jnp.float32),
        jax.ShapeDtypeStruct((NC, N_PAD), jnp.float32),
    ],
    mesh=_mesh,
    scratch_types=[
        pltpu.VMEM((NCHUNK, CH), jnp.int32),
        pltpu.VMEM((NCHUNK, CH), jnp.int32),
        pltpu.VMEM((CH,), jnp.float32),
        pltpu.VMEM((640,), jnp.float32),
        pltpu.VMEM_SHARED((N_PAD,), jnp.float32),
        pltpu.VMEM_SHARED((N_PAD,), jnp.float32),
    ],
)
def _degree_kernel(src_hbm, dst_hbm, outdeg_hbm, indeg_hbm,
                   sidx, didx, ones_v, zeros_v, acc_s, acc_d):
    cid = lax.axis_index("c")
    sid = lax.axis_index("s")
    wid = cid * NS + sid
    one16 = jnp.full((16,), 1.0, jnp.float32)
    zero16 = jnp.zeros((16,), jnp.float32)
    for j in range(CH // 16):
        ones_v[pl.ds(j * 16, 16)] = one16

    def _z(i, _):
        zeros_v[pl.ds(i * 16, 16)] = zero16
        return 0
    lax.fori_loop(0, _ROWS_PER_TILE // 16, _z, 0)

    base = sid * _ROWS_PER_TILE
    pltpu.sync_copy(zeros_v, acc_s.at[pl.ds(base, _ROWS_PER_TILE)])
    pltpu.sync_copy(zeros_v, acc_d.at[pl.ds(base, _ROWS_PER_TILE)])
    pltpu.sync_copy(src_hbm.at[wid], sidx)
    pltpu.sync_copy(dst_hbm.at[wid], didx)
    plsc.subcore_barrier()

    def _scatter(j, _):
        pltpu.sync_copy(ones_v, acc_s.at[sidx.at[j]], add=True)
        pltpu.sync_copy(ones_v, acc_d.at[didx.at[j]], add=True)
        return 0
    lax.fori_loop(0, NCHUNK, _scatter, 0)

    plsc.subcore_barrier()
    pltpu.sync_copy(acc_s.at[pl.ds(base, _ROWS_PER_TILE)],
                    outdeg_hbm.at[cid, pl.ds(base, _ROWS_PER_TILE)])
    pltpu.sync_copy(acc_d.at[pl.ds(base, _ROWS_PER_TILE)],
                    indeg_hbm.at[cid, pl.ds(base, _ROWS_PER_TILE)])


NB = 2
CH2 = 128
NSTAGE = 5
SCHUNK = 16


@functools.partial(
    pl.kernel,
    out_type=jax.ShapeDtypeStruct((NC, N_PAD, D), jnp.float32),
    mesh=_mesh,
    scratch_types=[
        pltpu.VMEM((SCHUNK, CH2), jnp.int32),
        pltpu.VMEM((SCHUNK, CH2), jnp.int32),
        pltpu.VMEM((NB, CH2, D), jnp.float32),
        pltpu.VMEM_SHARED((N_PAD, D), jnp.float32),
        pltpu.SemaphoreType.DMA,
        pltpu.SemaphoreType.DMA,
        pltpu.SemaphoreType.DMA,
        pltpu.SemaphoreType.DMA,
    ],
)
def _aggregate_kernel(feat_hbm, src_hbm, dst_hbm, part_hbm,
                      sidx, didx, rows, acc,
                      gs0, gs1, ss0, ss1):
    cid = lax.axis_index("c")
    sid = lax.axis_index("s")
    wid = cid * NS + sid
    gsems = (gs0, gs1)
    ssems = (ss0, ss1)
    zero16 = jnp.zeros((16,), jnp.float32)

    def _zrow(i, _):
        for j in range(D // 16):
            rows[0, i, pl.ds(j * 16, 16)] = zero16
        return 0
    lax.fori_loop(0, CH2, _zrow, 0)

    base = sid * _ROWS_PER_TILE
    for k in range(_ROWS_PER_TILE // CH2):
        pltpu.sync_copy(rows.at[0], acc.at[pl.ds(base + k * CH2, CH2)])
    plsc.subcore_barrier()

    def _gather(j, b):
        pltpu.async_copy(feat_hbm.at[sidx.at[j]], rows.at[b], gsems[b])

    def _wait_g(b):
        pltpu.make_async_copy(feat_hbm.at[sidx.at[0]], rows.at[b],
                              gsems[b]).wait()

    def _scatter(j, b):
        pltpu.async_copy(rows.at[b], acc.at[didx.at[j]], ssems[b], add=True)

    def _wait_s(b):
        pltpu.make_async_copy(rows.at[b], acc.at[didx.at[0]], ssems[b]).wait()

    def _stage(s, _):
        pltpu.sync_copy(src_hbm.at[wid, pl.ds(s * SCHUNK, SCHUNK)], sidx)
        pltpu.sync_copy(dst_hbm.at[wid, pl.ds(s * SCHUNK, SCHUNK)], didx)

        for j in range(NB - 1):
            _gather(j, j % NB)
        for j in range(SCHUNK):
            b = j % NB
            _wait_g(b)
            _scatter(j, b)
            nj = j + NB - 1
            if nj < SCHUNK:
                if j >= 1:
                    _wait_s(nj % NB)
                _gather(nj, nj % NB)
        for b in range(min(NB, SCHUNK)):
            _wait_s(b)
        return 0
    lax.fori_loop(0, NSTAGE, _stage, 0)

    plsc.subcore_barrier()
    pltpu.sync_copy(acc.at[pl.ds(base, _ROWS_PER_TILE)],
                    part_hbm.at[cid, pl.ds(base, _ROWS_PER_TILE)])


def _scale_body(x_ref, deg_ref, o_ref):
    dsum = deg_ref[0] + deg_ref[1]
    norm = lax.rsqrt(jnp.maximum(dsum, 1.0))
    o_ref[...] = x_ref[...] * norm


def _scale_features(x_pad, outdeg_p):
    blk = 1024
    grid = (N_PAD // blk,)
    return pl.pallas_call(
        _scale_body,
        grid=grid,
        in_specs=[
            pl.BlockSpec((blk, D), lambda i: (i, 0)),
            pl.BlockSpec((NC, blk, 1), lambda i: (0, i, 0)),
        ],
        out_specs=pl.BlockSpec((blk, D), lambda i: (i, 0)),
        out_shape=jax.ShapeDtypeStruct((N_PAD, D), jnp.float32),
    )(x_pad, outdeg_p)


def _final_body(part_ref, w_ref, m_ref, deg_ref, b_ref, o_ref):
    a = part_ref[0] + part_ref[1]
    w_t = jnp.where(m_ref[...] > THRESH, w_ref[...], 0.0)
    y = jnp.dot(a, w_t, preferred_element_type=jnp.float32)
    dsum = deg_ref[0] + deg_ref[1]
    norm = lax.rsqrt(jnp.maximum(dsum, 1.0))
    o_ref[...] = y * norm + b_ref[...]


def _finalize(part, weight, mask_real, indeg_p, bias):
    blk = 1024
    grid = (N_PAD // blk,)
    return pl.pallas_call(
        _final_body,
        grid=grid,
        in_specs=[
            pl.BlockSpec((NC, blk, D), lambda i: (0, i, 0)),
            pl.BlockSpec((D, D), lambda i: (0, 0)),
            pl.BlockSpec((D, D), lambda i: (0, 0)),
            pl.BlockSpec((NC, blk, 1), lambda i: (0, i, 0)),
            pl.BlockSpec((1, D), lambda i: (0, 0)),
        ],
        out_specs=pl.BlockSpec((blk, D), lambda i: (i, 0)),
        out_shape=jax.ShapeDtypeStruct((N_PAD, D), jnp.float32),
    )(part, weight, mask_real, indeg_p, bias)


def kernel(x, edge_index, weight, bias, mask_real):
    src = edge_index[0]
    dst = edge_index[1]
    n_pad_e = E_PAD - N_EDGES
    junk = N_NODES + (jnp.arange(n_pad_e, dtype=jnp.int32) % (N_PAD - N_NODES))
    src_flat = jnp.concatenate([src, junk])
    dst_flat = jnp.concatenate([dst, junk])
    x_pad = jnp.concatenate(
        [x, jnp.zeros((N_PAD - N_NODES, D), jnp.float32)], axis=0)

    outdeg_p, indeg_p = _degree_kernel(
        src_flat.reshape(NW, NCHUNK, CH), dst_flat.reshape(NW, NCHUNK, CH))
    feat = _scale_features(x_pad, outdeg_p.reshape(NC, N_PAD, 1))
    part = _aggregate_kernel(
        feat,
        src_flat.reshape(NW, NSTAGE * SCHUNK, CH2),
        dst_flat.reshape(NW, NSTAGE * SCHUNK, CH2))
    out = _finalize(part, weight, mask_real,
                    indeg_p.reshape(NC, N_PAD, 1), bias.reshape(1, D))
    return out[:N_NODES]


# speedup vs baseline: 1.0837x; 1.0837x over previous
import functools

import jax
import jax.numpy as jnp
from jax import lax
from jax.experimental import pallas as pl
from jax.experimental.pallas import tpu as pltpu
from jax.experimental.pallas import tpu_sc as plsc

N_NODES = 10000
N_EDGES = 320000
D = 128
THRESH = 0.005

NC = 2
NS = 16
NW = NC * NS
CH = 128
NCHUNK = 80
E_PAD = NW * NCHUNK * CH
N_PAD = 10240

_mesh = plsc.VectorSubcoreMesh(core_axis_name="c", subcore_axis_name="s")

_ROWS_PER_TILE = N_PAD // NS


@functools.partial(
    pl.kernel,
    out_type=[
        jax.ShapeDtypeStruct((NC, N_PAD), jnp.float32),
        jax.ShapeDtypeStruct((NC, N_PAD), jnp.float32),
    ],
    mesh=_mesh,
    scratch_types=[
        pltpu.VMEM((NCHUNK, CH), jnp.int32),
        pltpu.VMEM((NCHUNK, CH), jnp.int32),
        pltpu.VMEM((CH,), jnp.float32),
        pltpu.VMEM((640,), jnp.float32),
        pltpu.VMEM_SHARED((N_PAD,), jnp.float32),
        pltpu.VMEM_SHARED((N_PAD,), jnp.float32),
    ],
)
def _degree_kernel(src_hbm, dst_hbm, outdeg_hbm, indeg_hbm,
                   sidx, didx, ones_v, zeros_v, acc_s, acc_d):
    cid = lax.axis_index("c")
    sid = lax.axis_index("s")
    wid = cid * NS + sid
    one16 = jnp.full((16,), 1.0, jnp.float32)
    zero16 = jnp.zeros((16,), jnp.float32)
    for j in range(CH // 16):
        ones_v[pl.ds(j * 16, 16)] = one16

    def _z(i, _):
        zeros_v[pl.ds(i * 16, 16)] = zero16
        return 0
    lax.fori_loop(0, _ROWS_PER_TILE // 16, _z, 0)

    base = sid * _ROWS_PER_TILE
    pltpu.sync_copy(zeros_v, acc_s.at[pl.ds(base, _ROWS_PER_TILE)])
    pltpu.sync_copy(zeros_v, acc_d.at[pl.ds(base, _ROWS_PER_TILE)])
    pltpu.sync_copy(src_hbm.at[wid], sidx)
    pltpu.sync_copy(dst_hbm.at[wid], didx)
    plsc.subcore_barrier()

    def _scatter(j, _):
        pltpu.sync_copy(ones_v, acc_s.at[sidx.at[j]], add=True)
        pltpu.sync_copy(ones_v, acc_d.at[didx.at[j]], add=True)
        return 0
    lax.fori_loop(0, NCHUNK, _scatter, 0)

    plsc.subcore_barrier()
    pltpu.sync_copy(acc_s.at[pl.ds(base, _ROWS_PER_TILE)],
                    outdeg_hbm.at[cid, pl.ds(base, _ROWS_PER_TILE)])
    pltpu.sync_copy(acc_d.at[pl.ds(base, _ROWS_PER_TILE)],
                    indeg_hbm.at[cid, pl.ds(base, _ROWS_PER_TILE)])


NB = 5
CH2 = 64
NSTAGE = 10
SCHUNK = 16


@functools.partial(
    pl.kernel,
    out_type=jax.ShapeDtypeStruct((NC, N_PAD, D), jnp.float32),
    mesh=_mesh,
    scratch_types=[
        pltpu.VMEM((SCHUNK, CH2), jnp.int32),
        pltpu.VMEM((SCHUNK, CH2), jnp.int32),
        pltpu.VMEM((NB, CH2, D), jnp.float32),
        pltpu.VMEM_SHARED((N_PAD, D), jnp.float32),
        pltpu.SemaphoreType.DMA,
        pltpu.SemaphoreType.DMA,
        pltpu.SemaphoreType.DMA,
        pltpu.SemaphoreType.DMA,
        pltpu.SemaphoreType.DMA,
        pltpu.SemaphoreType.DMA,
        pltpu.SemaphoreType.DMA,
        pltpu.SemaphoreType.DMA,
        pltpu.SemaphoreType.DMA,
        pltpu.SemaphoreType.DMA,
    ],
)
def _aggregate_kernel(feat_hbm, src_hbm, dst_hbm, part_hbm,
                      sidx, didx, rows, acc, *sems):
    cid = lax.axis_index("c")
    sid = lax.axis_index("s")
    wid = cid * NS + sid
    gsems = sems[:NB]
    ssems = sems[NB:]
    zero16 = jnp.zeros((16,), jnp.float32)

    def _zrow(i, _):
        for j in range(D // 16):
            rows[0, i, pl.ds(j * 16, 16)] = zero16
        return 0
    lax.fori_loop(0, CH2, _zrow, 0)

    base = sid * _ROWS_PER_TILE
    for k in range(_ROWS_PER_TILE // CH2):
        pltpu.sync_copy(rows.at[0], acc.at[pl.ds(base + k * CH2, CH2)])
    plsc.subcore_barrier()

    def _gather(j, b):
        pltpu.async_copy(feat_hbm.at[sidx.at[j]], rows.at[b], gsems[b])

    def _wait_g(b):
        pltpu.make_async_copy(feat_hbm.at[sidx.at[0]], rows.at[b],
                              gsems[b]).wait()

    def _scatter(j, b):
        pltpu.async_copy(rows.at[b], acc.at[didx.at[j]], ssems[b], add=True)

    def _wait_s(b):
        pltpu.make_async_copy(rows.at[b], acc.at[didx.at[0]], ssems[b]).wait()

    def _stage(s, _):
        pltpu.sync_copy(src_hbm.at[wid, pl.ds(s * SCHUNK, SCHUNK)], sidx)
        pltpu.sync_copy(dst_hbm.at[wid, pl.ds(s * SCHUNK, SCHUNK)], didx)

        for j in range(NB - 1):
            _gather(j, j % NB)
        for j in range(SCHUNK):
            b = j % NB
            _wait_g(b)
            _scatter(j, b)
            nj = j + NB - 1
            if nj < SCHUNK:
                if j >= 1:
                    _wait_s(nj % NB)
                _gather(nj, nj % NB)
        for b in range(min(NB, SCHUNK)):
            _wait_s(b)
        return 0
    lax.fori_loop(0, NSTAGE, _stage, 0)

    plsc.subcore_barrier()
    pltpu.sync_copy(acc.at[pl.ds(base, _ROWS_PER_TILE)],
                    part_hbm.at[cid, pl.ds(base, _ROWS_PER_TILE)])


def _scale_body(x_ref, deg_ref, o_ref):
    dsum = deg_ref[0] + deg_ref[1]
    norm = lax.rsqrt(jnp.maximum(dsum, 1.0))
    o_ref[...] = x_ref[...] * norm


def _scale_features(x_pad, outdeg_p):
    blk = 1024
    grid = (N_PAD // blk,)
    return pl.pallas_call(
        _scale_body,
        grid=grid,
        in_specs=[
            pl.BlockSpec((blk, D), lambda i: (i, 0)),
            pl.BlockSpec((NC, blk, 1), lambda i: (0, i, 0)),
        ],
        out_specs=pl.BlockSpec((blk, D), lambda i: (i, 0)),
        out_shape=jax.ShapeDtypeStruct((N_PAD, D), jnp.float32),
    )(x_pad, outdeg_p)


def _final_body(part_ref, w_ref, m_ref, deg_ref, b_ref, o_ref):
    a = part_ref[0] + part_ref[1]
    w_t = jnp.where(m_ref[...] > THRESH, w_ref[...], 0.0)
    y = jnp.dot(a, w_t, preferred_element_type=jnp.float32)
    dsum = deg_ref[0] + deg_ref[1]
    norm = lax.rsqrt(jnp.maximum(dsum, 1.0))
    o_ref[...] = y * norm + b_ref[...]


def _finalize(part, weight, mask_real, indeg_p, bias):
    blk = 1024
    grid = (N_PAD // blk,)
    return pl.pallas_call(
        _final_body,
        grid=grid,
        in_specs=[
            pl.BlockSpec((NC, blk, D), lambda i: (0, i, 0)),
            pl.BlockSpec((D, D), lambda i: (0, 0)),
            pl.BlockSpec((D, D), lambda i: (0, 0)),
            pl.BlockSpec((NC, blk, 1), lambda i: (0, i, 0)),
            pl.BlockSpec((1, D), lambda i: (0, 0)),
        ],
        out_specs=pl.BlockSpec((blk, D), lambda i: (i, 0)),
        out_shape=jax.ShapeDtypeStruct((N_PAD, D), jnp.float32),
    )(part, weight, mask_real, indeg_p, bias)


def kernel(x, edge_index, weight, bias, mask_real):
    src = edge_index[0]
    dst = edge_index[1]
    n_pad_e = E_PAD - N_EDGES
    junk = N_NODES + (jnp.arange(n_pad_e, dtype=jnp.int32) % (N_PAD - N_NODES))
    src_flat = jnp.concatenate([src, junk])
    dst_flat = jnp.concatenate([dst, junk])
    x_pad = jnp.concatenate(
        [x, jnp.zeros((N_PAD - N_NODES, D), jnp.float32)], axis=0)

    outdeg_p, indeg_p = _degree_kernel(
        src_flat.reshape(NW, NCHUNK, CH), dst_flat.reshape(NW, NCHUNK, CH))
    feat = _scale_features(x_pad, outdeg_p.reshape(NC, N_PAD, 1))
    part = _aggregate_kernel(
        feat,
        src_flat.reshape(NW, NSTAGE * SCHUNK, CH2),
        dst_flat.reshape(NW, NSTAGE * SCHUNK, CH2))
    out = _finalize(part, weight, mask_real,
                    indeg_p.reshape(NC, N_PAD, 1), bias.reshape(1, D))
    return out[:N_NODES]


# speedup vs baseline: 1.1213x; 1.0347x over previous
import functools

import jax
import jax.numpy as jnp
from jax import lax
from jax.experimental import pallas as pl
from jax.experimental.pallas import tpu as pltpu
from jax.experimental.pallas import tpu_sc as plsc

N_NODES = 10000
N_EDGES = 320000
D = 128
THRESH = 0.005

NC = 2
NS = 16
NW = NC * NS
CH = 128
NCHUNK = 80
E_PAD = NW * NCHUNK * CH
N_PAD = 10240

_mesh = plsc.VectorSubcoreMesh(core_axis_name="c", subcore_axis_name="s")

_ROWS_PER_TILE = N_PAD // NS


@functools.partial(
    pl.kernel,
    out_type=[
        jax.ShapeDtypeStruct((NC, N_PAD), jnp.float32),
        jax.ShapeDtypeStruct((NC, N_PAD), jnp.float32),
    ],
    mesh=_mesh,
    scratch_types=[
        pltpu.VMEM((NCHUNK, CH), jnp.int32),
        pltpu.VMEM((NCHUNK, CH), jnp.int32),
        pltpu.VMEM((CH,), jnp.float32),
        pltpu.VMEM((640,), jnp.float32),
        pltpu.VMEM_SHARED((N_PAD,), jnp.float32),
        pltpu.VMEM_SHARED((N_PAD,), jnp.float32),
    ],
)
def _degree_kernel(src_hbm, dst_hbm, outdeg_hbm, indeg_hbm,
                   sidx, didx, ones_v, zeros_v, acc_s, acc_d):
    cid = lax.axis_index("c")
    sid = lax.axis_index("s")
    wid = cid * NS + sid
    one16 = jnp.full((16,), 1.0, jnp.float32)
    zero16 = jnp.zeros((16,), jnp.float32)
    for j in range(CH // 16):
        ones_v[pl.ds(j * 16, 16)] = one16

    def _z(i, _):
        zeros_v[pl.ds(i * 16, 16)] = zero16
        return 0
    lax.fori_loop(0, _ROWS_PER_TILE // 16, _z, 0)

    base = sid * _ROWS_PER_TILE
    pltpu.sync_copy(zeros_v, acc_s.at[pl.ds(base, _ROWS_PER_TILE)])
    pltpu.sync_copy(zeros_v, acc_d.at[pl.ds(base, _ROWS_PER_TILE)])
    pltpu.sync_copy(src_hbm.at[wid], sidx)
    pltpu.sync_copy(dst_hbm.at[wid], didx)
    plsc.subcore_barrier()

    def _scatter(j, _):
        pltpu.sync_copy(ones_v, acc_s.at[sidx.at[j]], add=True)
        pltpu.sync_copy(ones_v, acc_d.at[didx.at[j]], add=True)
        return 0
    lax.fori_loop(0, NCHUNK, _scatter, 0)

    plsc.subcore_barrier()
    pltpu.sync_copy(acc_s.at[pl.ds(base, _ROWS_PER_TILE)],
                    outdeg_hbm.at[cid, pl.ds(base, _ROWS_PER_TILE)])
    pltpu.sync_copy(acc_d.at[pl.ds(base, _ROWS_PER_TILE)],
                    indeg_hbm.at[cid, pl.ds(base, _ROWS_PER_TILE)])


NB = 4
CH2 = 64
NSTAGE = 4
SCHUNK = 40


@functools.partial(
    pl.kernel,
    out_type=jax.ShapeDtypeStruct((NC, N_PAD, D), jnp.float32),
    mesh=_mesh,
    scratch_types=[
        pltpu.VMEM((SCHUNK, CH2), jnp.int32),
        pltpu.VMEM((SCHUNK, CH2), jnp.int32),
        pltpu.VMEM((NB, CH2, D), jnp.float32),
        pltpu.VMEM_SHARED((N_PAD, D), jnp.float32),
        pltpu.SemaphoreType.DMA,
        pltpu.SemaphoreType.DMA,
        pltpu.SemaphoreType.DMA,
        pltpu.SemaphoreType.DMA,
        pltpu.SemaphoreType.DMA,
        pltpu.SemaphoreType.DMA,
        pltpu.SemaphoreType.DMA,
        pltpu.SemaphoreType.DMA,
        pltpu.SemaphoreType.DMA,
        pltpu.SemaphoreType.DMA,
    ],
)
def _aggregate_kernel(feat_hbm, src_hbm, dst_hbm, part_hbm,
                      sidx, didx, rows, acc, *sems):
    cid = lax.axis_index("c")
    sid = lax.axis_index("s")
    wid = cid * NS + sid
    gsems = sems[:NB]
    ssems = sems[NB:]
    zero16 = jnp.zeros((16,), jnp.float32)

    def _zrow(i, _):
        for j in range(D // 16):
            rows[0, i, pl.ds(j * 16, 16)] = zero16
        return 0
    lax.fori_loop(0, CH2, _zrow, 0)

    base = sid * _ROWS_PER_TILE
    for k in range(_ROWS_PER_TILE // CH2):
        pltpu.sync_copy(rows.at[0], acc.at[pl.ds(base + k * CH2, CH2)])
    plsc.subcore_barrier()

    def _gather(j, b):
        pltpu.async_copy(feat_hbm.at[sidx.at[j]], rows.at[b], gsems[b])

    def _wait_g(b):
        pltpu.make_async_copy(feat_hbm.at[sidx.at[0]], rows.at[b],
                              gsems[b]).wait()

    def _scatter(j, b):
        pltpu.async_copy(rows.at[b], acc.at[didx.at[j]], ssems[b], add=True)

    def _wait_s(b):
        pltpu.make_async_copy(rows.at[b], acc.at[didx.at[0]], ssems[b]).wait()

    def _stage(s, _):
        pltpu.sync_copy(src_hbm.at[wid, pl.ds(s * SCHUNK, SCHUNK)], sidx)
        pltpu.sync_copy(dst_hbm.at[wid, pl.ds(s * SCHUNK, SCHUNK)], didx)

        for j in range(NB - 1):
            _gather(j, j % NB)
        for j in range(SCHUNK):
            b = j % NB
            _wait_g(b)
            _scatter(j, b)
            nj = j + NB - 1
            if nj < SCHUNK:
                if j >= 1:
                    _wait_s(nj % NB)
                _gather(nj, nj % NB)
        for b in range(min(NB, SCHUNK)):
            _wait_s(b)
        return 0
    lax.fori_loop(0, NSTAGE, _stage, 0)

    plsc.subcore_barrier()
    pltpu.sync_copy(acc.at[pl.ds(base, _ROWS_PER_TILE)],
                    part_hbm.at[cid, pl.ds(base, _ROWS_PER_TILE)])


def _scale_body(x_ref, deg_ref, o_ref):
    dsum = deg_ref[0] + deg_ref[1]
    norm = lax.rsqrt(jnp.maximum(dsum, 1.0))
    o_ref[...] = x_ref[...] * norm


def _scale_features(x_pad, outdeg_p):
    blk = 1024
    grid = (N_PAD // blk,)
    return pl.pallas_call(
        _scale_body,
        grid=grid,
        in_specs=[
            pl.BlockSpec((blk, D), lambda i: (i, 0)),
            pl.BlockSpec((NC, blk, 1), lambda i: (0, i, 0)),
        ],
        out_specs=pl.BlockSpec((blk, D), lambda i: (i, 0)),
        out_shape=jax.ShapeDtypeStruct((N_PAD, D), jnp.float32),
    )(x_pad, outdeg_p)


def _final_body(part_ref, w_ref, m_ref, deg_ref, b_ref, o_ref):
    a = part_ref[0] + part_ref[1]
    w_t = jnp.where(m_ref[...] > THRESH, w_ref[...], 0.0)
    y = jnp.dot(a, w_t, preferred_element_type=jnp.float32)
    dsum = deg_ref[0] + deg_ref[1]
    norm = lax.rsqrt(jnp.maximum(dsum, 1.0))
    o_ref[...] = y * norm + b_ref[...]


def _finalize(part, weight, mask_real, indeg_p, bias):
    blk = 1024
    grid = (N_PAD // blk,)
    return pl.pallas_call(
        _final_body,
        grid=grid,
        in_specs=[
            pl.BlockSpec((NC, blk, D), lambda i: (0, i, 0)),
            pl.BlockSpec((D, D), lambda i: (0, 0)),
            pl.BlockSpec((D, D), lambda i: (0, 0)),
            pl.BlockSpec((NC, blk, 1), lambda i: (0, i, 0)),
            pl.BlockSpec((1, D), lambda i: (0, 0)),
        ],
        out_specs=pl.BlockSpec((blk, D), lambda i: (i, 0)),
        out_shape=jax.ShapeDtypeStruct((N_PAD, D), jnp.float32),
    )(part, weight, mask_real, indeg_p, bias)


def kernel(x, edge_index, weight, bias, mask_real):
    src = edge_index[0]
    dst = edge_index[1]
    n_pad_e = E_PAD - N_EDGES
    junk = N_NODES + (jnp.arange(n_pad_e, dtype=jnp.int32) % (N_PAD - N_NODES))
    src_flat = jnp.concatenate([src, junk])
    dst_flat = jnp.concatenate([dst, junk])
    x_pad = jnp.concatenate(
        [x, jnp.zeros((N_PAD - N_NODES, D), jnp.float32)], axis=0)

    outdeg_p, indeg_p = _degree_kernel(
        src_flat.reshape(NW, NCHUNK, CH), dst_flat.reshape(NW, NCHUNK, CH))
    feat = _scale_features(x_pad, outdeg_p.reshape(NC, N_PAD, 1))
    part = _aggregate_kernel(
        feat,
        src_flat.reshape(NW, NSTAGE * SCHUNK, CH2),
        dst_flat.reshape(NW, NSTAGE * SCHUNK, CH2))
    out = _finalize(part, weight, mask_real,
                    indeg_p.reshape(NC, N_PAD, 1), bias.reshape(1, D))
    return out[:N_NODES]


# speedup vs baseline: 1.1694x; 1.0429x over previous
import functools

import jax
import jax.numpy as jnp
from jax import lax
from jax.experimental import pallas as pl
from jax.experimental.pallas import tpu as pltpu
from jax.experimental.pallas import tpu_sc as plsc

N_NODES = 10000
N_EDGES = 320000
D = 128
THRESH = 0.005

NC = 2
NS = 16
NW = NC * NS
CH = 128
NCHUNK = 80
E_PAD = NW * NCHUNK * CH
N_PAD = 10240

_mesh = plsc.VectorSubcoreMesh(core_axis_name="c", subcore_axis_name="s")

_ROWS_PER_TILE = N_PAD // NS


@functools.partial(
    pl.kernel,
    out_type=[
        jax.ShapeDtypeStruct((NC, N_PAD), jnp.float32),
        jax.ShapeDtypeStruct((NC, N_PAD), jnp.float32),
    ],
    mesh=_mesh,
    scratch_types=[
        pltpu.VMEM((NCHUNK, CH), jnp.int32),
        pltpu.VMEM((NCHUNK, CH), jnp.int32),
        pltpu.VMEM((CH,), jnp.float32),
        pltpu.VMEM((640,), jnp.float32),
        pltpu.VMEM_SHARED((N_PAD,), jnp.float32),
        pltpu.VMEM_SHARED((N_PAD,), jnp.float32),
        pltpu.SemaphoreType.DMA,
        pltpu.SemaphoreType.DMA,
    ],
)
def _degree_kernel(src_hbm, dst_hbm, outdeg_hbm, indeg_hbm,
                   sidx, didx, ones_v, zeros_v, acc_s, acc_d, sem_s, sem_d):
    cid = lax.axis_index("c")
    sid = lax.axis_index("s")
    wid = cid * NS + sid
    one16 = jnp.full((16,), 1.0, jnp.float32)
    zero16 = jnp.zeros((16,), jnp.float32)
    for j in range(CH // 16):
        ones_v[pl.ds(j * 16, 16)] = one16

    def _z(i, _):
        zeros_v[pl.ds(i * 16, 16)] = zero16
        return 0
    lax.fori_loop(0, _ROWS_PER_TILE // 16, _z, 0)

    base = sid * _ROWS_PER_TILE
    pltpu.sync_copy(zeros_v, acc_s.at[pl.ds(base, _ROWS_PER_TILE)])
    pltpu.sync_copy(zeros_v, acc_d.at[pl.ds(base, _ROWS_PER_TILE)])
    pltpu.sync_copy(src_hbm.at[wid], sidx)
    pltpu.sync_copy(dst_hbm.at[wid], didx)
    plsc.subcore_barrier()

    def _scatter(j, _):
        pltpu.async_copy(ones_v, acc_s.at[sidx.at[j]], sem_s, add=True)
        pltpu.async_copy(ones_v, acc_d.at[didx.at[j]], sem_d, add=True)
        return 0
    lax.fori_loop(0, NCHUNK, _scatter, 0)

    def _drain(j, _):
        pltpu.make_async_copy(ones_v, acc_s.at[sidx.at[0]], sem_s).wait()
        pltpu.make_async_copy(ones_v, acc_d.at[didx.at[0]], sem_d).wait()
        return 0
    lax.fori_loop(0, NCHUNK, _drain, 0)

    plsc.subcore_barrier()
    pltpu.sync_copy(acc_s.at[pl.ds(base, _ROWS_PER_TILE)],
                    outdeg_hbm.at[cid, pl.ds(base, _ROWS_PER_TILE)])
    pltpu.sync_copy(acc_d.at[pl.ds(base, _ROWS_PER_TILE)],
                    indeg_hbm.at[cid, pl.ds(base, _ROWS_PER_TILE)])


NB = 4
CH2 = 64
NSTAGE = 4
SCHUNK = 40


@functools.partial(
    pl.kernel,
    out_type=jax.ShapeDtypeStruct((NC, N_PAD, D), jnp.float32),
    mesh=_mesh,
    scratch_types=[
        pltpu.VMEM((SCHUNK, CH2), jnp.int32),
        pltpu.VMEM((SCHUNK, CH2), jnp.int32),
        pltpu.VMEM((NB, CH2, D), jnp.float32),
        pltpu.VMEM_SHARED((N_PAD, D), jnp.float32),
        pltpu.SemaphoreType.DMA,
        pltpu.SemaphoreType.DMA,
        pltpu.SemaphoreType.DMA,
        pltpu.SemaphoreType.DMA,
        pltpu.SemaphoreType.DMA,
        pltpu.SemaphoreType.DMA,
        pltpu.SemaphoreType.DMA,
        pltpu.SemaphoreType.DMA,
        pltpu.SemaphoreType.DMA,
        pltpu.SemaphoreType.DMA,
    ],
)
def _aggregate_kernel(feat_hbm, src_hbm, dst_hbm, part_hbm,
                      sidx, didx, rows, acc, *sems):
    cid = lax.axis_index("c")
    sid = lax.axis_index("s")
    wid = cid * NS + sid
    gsems = sems[:NB]
    ssems = sems[NB:]
    zero16 = jnp.zeros((16,), jnp.float32)

    def _zrow(i, _):
        for j in range(D // 16):
            rows[0, i, pl.ds(j * 16, 16)] = zero16
        return 0
    lax.fori_loop(0, CH2, _zrow, 0)

    base = sid * _ROWS_PER_TILE
    for k in range(_ROWS_PER_TILE // CH2):
        pltpu.sync_copy(rows.at[0], acc.at[pl.ds(base + k * CH2, CH2)])
    plsc.subcore_barrier()

    def _gather(j, b):
        pltpu.async_copy(feat_hbm.at[sidx.at[j]], rows.at[b], gsems[b])

    def _wait_g(b):
        pltpu.make_async_copy(feat_hbm.at[sidx.at[0]], rows.at[b],
                              gsems[b]).wait()

    def _scatter(j, b):
        pltpu.async_copy(rows.at[b], acc.at[didx.at[j]], ssems[b], add=True)

    def _wait_s(b):
        pltpu.make_async_copy(rows.at[b], acc.at[didx.at[0]], ssems[b]).wait()

    def _stage(s, _):
        pltpu.sync_copy(src_hbm.at[wid, pl.ds(s * SCHUNK, SCHUNK)], sidx)
        pltpu.sync_copy(dst_hbm.at[wid, pl.ds(s * SCHUNK, SCHUNK)], didx)

        for j in range(NB - 1):
            _gather(j, j % NB)
        for j in range(SCHUNK):
            b = j % NB
            _wait_g(b)
            _scatter(j, b)
            nj = j + NB - 1
            if nj < SCHUNK:
                if j >= 1:
                    _wait_s(nj % NB)
                _gather(nj, nj % NB)
        for b in range(min(NB, SCHUNK)):
            _wait_s(b)
        return 0
    lax.fori_loop(0, NSTAGE, _stage, 0)

    plsc.subcore_barrier()
    pltpu.sync_copy(acc.at[pl.ds(base, _ROWS_PER_TILE)],
                    part_hbm.at[cid, pl.ds(base, _ROWS_PER_TILE)])


def _scale_body(x_ref, deg_ref, o_ref):
    dsum = deg_ref[0] + deg_ref[1]
    norm = lax.rsqrt(jnp.maximum(dsum, 1.0))
    o_ref[...] = x_ref[...] * norm


def _scale_features(x_pad, outdeg_p):
    blk = 1024
    grid = (N_PAD // blk,)
    return pl.pallas_call(
        _scale_body,
        grid=grid,
        in_specs=[
            pl.BlockSpec((blk, D), lambda i: (i, 0)),
            pl.BlockSpec((NC, blk, 1), lambda i: (0, i, 0)),
        ],
        out_specs=pl.BlockSpec((blk, D), lambda i: (i, 0)),
        out_shape=jax.ShapeDtypeStruct((N_PAD, D), jnp.float32),
    )(x_pad, outdeg_p)


def _final_body(part_ref, w_ref, m_ref, deg_ref, b_ref, o_ref):
    a = part_ref[0] + part_ref[1]
    w_t = jnp.where(m_ref[...] > THRESH, w_ref[...], 0.0)
    y = jnp.dot(a, w_t, preferred_element_type=jnp.float32)
    dsum = deg_ref[0] + deg_ref[1]
    norm = lax.rsqrt(jnp.maximum(dsum, 1.0))
    o_ref[...] = y * norm + b_ref[...]


def _finalize(part, weight, mask_real, indeg_p, bias):
    blk = 1024
    grid = (N_PAD // blk,)
    return pl.pallas_call(
        _final_body,
        grid=grid,
        in_specs=[
            pl.BlockSpec((NC, blk, D), lambda i: (0, i, 0)),
            pl.BlockSpec((D, D), lambda i: (0, 0)),
            pl.BlockSpec((D, D), lambda i: (0, 0)),
            pl.BlockSpec((NC, blk, 1), lambda i: (0, i, 0)),
            pl.BlockSpec((1, D), lambda i: (0, 0)),
        ],
        out_specs=pl.BlockSpec((blk, D), lambda i: (i, 0)),
        out_shape=jax.ShapeDtypeStruct((N_PAD, D), jnp.float32),
    )(part, weight, mask_real, indeg_p, bias)


def kernel(x, edge_index, weight, bias, mask_real):
    src = edge_index[0]
    dst = edge_index[1]
    n_pad_e = E_PAD - N_EDGES
    junk = N_NODES + (jnp.arange(n_pad_e, dtype=jnp.int32) % (N_PAD - N_NODES))
    src_flat = jnp.concatenate([src, junk])
    dst_flat = jnp.concatenate([dst, junk])
    x_pad = jnp.concatenate(
        [x, jnp.zeros((N_PAD - N_NODES, D), jnp.float32)], axis=0)

    outdeg_p, indeg_p = _degree_kernel(
        src_flat.reshape(NW, NCHUNK, CH), dst_flat.reshape(NW, NCHUNK, CH))
    feat = _scale_features(x_pad, outdeg_p.reshape(NC, N_PAD, 1))
    part = _aggregate_kernel(
        feat,
        src_flat.reshape(NW, NSTAGE * SCHUNK, CH2),
        dst_flat.reshape(NW, NSTAGE * SCHUNK, CH2))
    out = _finalize(part, weight, mask_real,
                    indeg_p.reshape(NC, N_PAD, 1), bias.reshape(1, D))
    return out[:N_NODES]
